# baseline (speedup 1.0000x reference)
import jax
import jax.numpy as jnp
from jax.experimental import pallas as pl
from jax.experimental.pallas import tpu as pltpu

_NUM_SAMPLES = 16


def _dma_gather(idx_ref, x_hbm, o_hbm, sem):
    b = x_hbm.shape[0]
    copies = []
    for i in range(b):
        for s in range(_NUM_SAMPLES):
            t = idx_ref[s]
            c = pltpu.make_async_copy(x_hbm.at[i, t], o_hbm.at[i, s], sem)
            c.start()
            copies.append(c)
    for c in copies:
        c.wait()


@jax.jit
def kernel(x):
    b, t, c, h, w = x.shape
    idx = jnp.linspace(0.0, float(t - 1), _NUM_SAMPLES).astype(jnp.int32)
    out = pl.pallas_call(
        _dma_gather,
        grid_spec=pltpu.PrefetchScalarGridSpec(
            num_scalar_prefetch=1,
            grid=(1,),
            in_specs=[pl.BlockSpec(memory_space=pl.ANY)],
            out_specs=pl.BlockSpec(memory_space=pl.ANY),
            scratch_shapes=[pltpu.SemaphoreType.DMA],
        ),
        out_shape=jax.ShapeDtypeStruct((b, _NUM_SAMPLES, c, h, w), x.dtype),
    )(idx, x)
    return out


# speedup vs baseline: 40.8157x; 40.8157x over previous
import jax
import jax.numpy as jnp
from jax.experimental import pallas as pl
from jax.experimental.pallas import tpu as pltpu

_NUM_SAMPLES = 16
_NBUF = 8
_STAGGER = 3


def _dma_pipeline(idx_ref, x_hbm, o_hbm, bufs, in_sems, out_sems):
    b = x_hbm.shape[0]
    n = b * _NUM_SAMPLES
    in_copies = [None] * n
    out_copies = [None] * n

    def start_out(j):
        bi, si = divmod(j, _NUM_SAMPLES)
        k = j % _NBUF
        in_copies[j].wait()
        out_copies[j] = pltpu.make_async_copy(
            bufs.at[k], o_hbm.at[bi, si], out_sems.at[k])
        out_copies[j].start()

    for j in range(n):
        bi, si = divmod(j, _NUM_SAMPLES)
        k = j % _NBUF
        if j >= _NBUF:
            out_copies[j - _NBUF].wait()
        t = idx_ref[si]
        in_copies[j] = pltpu.make_async_copy(
            x_hbm.at[bi, t], bufs.at[k], in_sems.at[k])
        in_copies[j].start()
        if j >= _STAGGER:
            start_out(j - _STAGGER)
    for j in range(n - _STAGGER, n):
        start_out(j)
    for j in range(n - _NBUF, n):
        out_copies[j].wait()


@jax.jit
def kernel(x):
    b, t, c, h, w = x.shape
    idx = jnp.linspace(0.0, float(t - 1), _NUM_SAMPLES).astype(jnp.int32)
    out = pl.pallas_call(
        _dma_pipeline,
        grid_spec=pltpu.PrefetchScalarGridSpec(
            num_scalar_prefetch=1,
            grid=(1,),
            in_specs=[pl.BlockSpec(memory_space=pl.ANY)],
            out_specs=pl.BlockSpec(memory_space=pl.ANY),
            scratch_shapes=[
                pltpu.VMEM((_NBUF, c, h, w), x.dtype),
                pltpu.SemaphoreType.DMA((_NBUF,)),
                pltpu.SemaphoreType.DMA((_NBUF,)),
            ],
        ),
        out_shape=jax.ShapeDtypeStruct((b, _NUM_SAMPLES, c, h, w), x.dtype),
    )(idx, x)
    return out


# speedup vs baseline: 44.1364x; 1.0814x over previous
import jax
import jax.numpy as jnp
from jax.experimental import pallas as pl
from jax.experimental.pallas import tpu as pltpu

_NUM_SAMPLES = 16
_NBUF = 12
_STAGGER = 5


def _dma_pipeline(idx_ref, x_hbm, o_hbm, bufs, in_sems, out_sems):
    b = x_hbm.shape[0]
    n = b * _NUM_SAMPLES
    in_copies = [None] * n
    out_copies = [None] * n

    def start_out(j):
        bi, si = divmod(j, _NUM_SAMPLES)
        k = j % _NBUF
        in_copies[j].wait()
        out_copies[j] = pltpu.make_async_copy(
            bufs.at[k], o_hbm.at[bi, si], out_sems.at[k])
        out_copies[j].start()

    for j in range(n):
        bi, si = divmod(j, _NUM_SAMPLES)
        k = j % _NBUF
        if j >= _NBUF:
            out_copies[j - _NBUF].wait()
        t = idx_ref[si]
        in_copies[j] = pltpu.make_async_copy(
            x_hbm.at[bi, t], bufs.at[k], in_sems.at[k])
        in_copies[j].start()
        if j >= _STAGGER:
            start_out(j - _STAGGER)
    for j in range(n - _STAGGER, n):
        start_out(j)
    for j in range(n - _NBUF, n):
        out_copies[j].wait()


@jax.jit
def kernel(x):
    b, t, c, h, w = x.shape
    idx = jnp.linspace(0.0, float(t - 1), _NUM_SAMPLES).astype(jnp.int32)
    out = pl.pallas_call(
        _dma_pipeline,
        grid_spec=pltpu.PrefetchScalarGridSpec(
            num_scalar_prefetch=1,
            grid=(1,),
            in_specs=[pl.BlockSpec(memory_space=pl.ANY)],
            out_specs=pl.BlockSpec(memory_space=pl.ANY),
            scratch_shapes=[
                pltpu.VMEM((_NBUF, c, h, w), x.dtype),
                pltpu.SemaphoreType.DMA((_NBUF,)),
                pltpu.SemaphoreType.DMA((_NBUF,)),
            ],
        ),
        out_shape=jax.ShapeDtypeStruct((b, _NUM_SAMPLES, c, h, w), x.dtype),
    )(idx, x)
    return out


# speedup vs baseline: 45.2180x; 1.0245x over previous
import jax
import jax.numpy as jnp
from jax.experimental import pallas as pl
from jax.experimental.pallas import tpu as pltpu

_NUM_SAMPLES = 16
_NBUF = 16
_STAGGER = 7


def _dma_pipeline(idx_ref, x_hbm, o_hbm, bufs, in_sems, out_sems):
    b = x_hbm.shape[0]
    n = b * _NUM_SAMPLES
    in_copies = [None] * n
    out_copies = [None] * n

    def start_out(j):
        bi, si = divmod(j, _NUM_SAMPLES)
        k = j % _NBUF
        in_copies[j].wait()
        out_copies[j] = pltpu.make_async_copy(
            bufs.at[k], o_hbm.at[bi, si], out_sems.at[k])
        out_copies[j].start()

    for j in range(n):
        bi, si = divmod(j, _NUM_SAMPLES)
        k = j % _NBUF
        if j >= _NBUF:
            out_copies[j - _NBUF].wait()
        t = idx_ref[si]
        in_copies[j] = pltpu.make_async_copy(
            x_hbm.at[bi, t], bufs.at[k], in_sems.at[k])
        in_copies[j].start()
        if j >= _STAGGER:
            start_out(j - _STAGGER)
    for j in range(n - _STAGGER, n):
        start_out(j)
    for j in range(n - _NBUF, n):
        out_copies[j].wait()


@jax.jit
def kernel(x):
    b, t, c, h, w = x.shape
    idx = jnp.linspace(0.0, float(t - 1), _NUM_SAMPLES).astype(jnp.int32)
    out = pl.pallas_call(
        _dma_pipeline,
        grid_spec=pltpu.PrefetchScalarGridSpec(
            num_scalar_prefetch=1,
            grid=(1,),
            in_specs=[pl.BlockSpec(memory_space=pl.ANY)],
            out_specs=pl.BlockSpec(memory_space=pl.ANY),
            scratch_shapes=[
                pltpu.VMEM((_NBUF, c, h, w), x.dtype),
                pltpu.SemaphoreType.DMA((_NBUF,)),
                pltpu.SemaphoreType.DMA((_NBUF,)),
            ],
        ),
        out_shape=jax.ShapeDtypeStruct((b, _NUM_SAMPLES, c, h, w), x.dtype),
    )(idx, x)
    return out
